# baseline (speedup 1.0000x reference)
import functools

import jax
import jax.numpy as jnp
from jax import lax
from jax.experimental import pallas as pl
from jax.experimental.pallas import tpu as pltpu
from jax.experimental.pallas import tpu_sc as plsc

T = 128
N = 512
EMB = 16
PE_DIM = 128
OUT_DIM = PE_DIM + EMB
B_FIX = 8

PE_CHUNK = 64
SEG = 16


def _pe_table_block(pe_ref):
    c = jax.lax.broadcasted_iota(jnp.int32, (T, PE_DIM), 1)
    t = jax.lax.broadcasted_iota(jnp.int32, (T, PE_DIM), 0).astype(jnp.float32)
    pair = (c // 2).astype(jnp.float32)
    div = jnp.exp(pair * (-2.0 * jnp.log(10000.0) / PE_DIM))
    ang = t * div
    pe_ref[...] = jnp.where(c % 2 == 0, jnp.sin(ang), jnp.cos(ang))


def _pe_table():
    return pl.pallas_call(
        _pe_table_block,
        out_shape=jax.ShapeDtypeStruct((T, PE_DIM), jnp.float32),
    )()


def _sc_kernel(pe_hbm, wt_hbm, out_hbm, pe_v, wt_v, bufa, bufb, sem, semw):
    info = plsc.get_sparse_core_info()
    ncores = info.num_cores
    nw = ncores * info.num_subcores
    t_per_w = T // nw
    n_chunks = t_per_w * (PE_DIM // PE_CHUNK)
    wid = lax.axis_index("s") * ncores + lax.axis_index("c")
    t_base = wid * t_per_w
    zero = wid - wid

    win = (t_base // 8) * 8
    pltpu.sync_copy(pe_hbm.at[pl.ds(win, 8)], pe_v)
    pltpu.sync_copy(wt_hbm, wt_v)

    bufs = (bufa, bufb)

    def fill(ci, buf):
        row = t_base - win + ci // 2
        c0 = (ci % 2) * PE_CHUNK

        def group(g, _):
            vec = pe_v[row, pl.ds(c0 + g * SEG, SEG)]
            for j in range(SEG):
                val = jnp.broadcast_to(vec[j], (SEG,))
                c = g * SEG + j

                def seg(k, carry, c=c, val=val):
                    buf[c, pl.ds(k * SEG, SEG)] = val
                    return carry

                lax.fori_loop(zero, N // SEG + zero, seg, 0)
            return _

        lax.fori_loop(zero, PE_CHUNK // SEG + zero, group, 0)

    def chunk_dmas(ci, buf):
        t_glob = t_base + ci // 2
        c0 = (ci % 2) * PE_CHUNK
        grp = []
        for b in range(B_FIX):
            grp.append(
                pltpu.make_async_copy(
                    buf, out_hbm.at[b, t_glob, pl.ds(c0, PE_CHUNK)], sem
                )
            )
            if ci % 2 == 0:
                grp.append(
                    pltpu.make_async_copy(
                        wt_v, out_hbm.at[b, t_glob, pl.ds(PE_DIM, EMB)], semw
                    )
                )
        return grp

    fill(0, bufs[0])
    pending = []
    for ci in range(n_chunks):
        cur = bufs[ci % 2]
        grp = chunk_dmas(ci, cur)
        for cp in grp:
            cp.start()
        for cp in pending:
            cp.wait()
        if ci + 1 < n_chunks:
            fill(ci + 1, bufs[(ci + 1) % 2])
        pending = grp
    for cp in pending:
        cp.wait()


def kernel(u, W):
    batch = u.shape[0]
    pe = _pe_table()
    mesh = plsc.VectorSubcoreMesh(core_axis_name="c", subcore_axis_name="s")
    sck = functools.partial(
        pl.kernel,
        mesh=mesh,
        out_type=jax.ShapeDtypeStruct((batch, T, OUT_DIM, N), jnp.float32),
        scratch_types=[
            pltpu.VMEM((8, PE_DIM), jnp.float32),
            pltpu.VMEM((EMB, N), jnp.float32),
            pltpu.VMEM((PE_CHUNK, N), jnp.float32),
            pltpu.VMEM((PE_CHUNK, N), jnp.float32),
            pltpu.SemaphoreType.DMA,
            pltpu.SemaphoreType.DMA,
        ],
    )(_sc_kernel)
    out = sck(pe, W.T)
    return jnp.transpose(out, (0, 1, 3, 2))


# speedup vs baseline: 1.3514x; 1.3514x over previous
import jax
import jax.numpy as jnp
from jax.experimental import pallas as pl
from jax.experimental.pallas import tpu as pltpu

T = 128
N = 512
EMB = 16
PE_DIM = 128
OUT_DIM = PE_DIM + EMB
TC_CHUNK = 16
NBUF = 4


def _fill(buf, wt, tci):
    t0 = (tci * TC_CHUNK).astype(jnp.float32)
    c = jax.lax.broadcasted_iota(jnp.int32, (PE_DIM, TC_CHUNK), 0)
    t = t0 + jax.lax.broadcasted_iota(
        jnp.int32, (PE_DIM, TC_CHUNK), 1
    ).astype(jnp.float32)
    pair = (c // 2).astype(jnp.float32)
    div = jnp.exp(pair * (-2.0 * jnp.log(10000.0) / PE_DIM))
    ang = t * div
    val = jnp.where(c % 2 == 0, jnp.sin(ang), jnp.cos(ang))
    for ti in range(TC_CHUNK):
        buf[ti, 0:PE_DIM, :] = jnp.broadcast_to(
            val[:, ti : ti + 1], (PE_DIM, N)
        )
        buf[ti, PE_DIM:OUT_DIM, :] = wt


def _side_info_manual(w_ref, out_ref, bufs, sems):
    i = pl.program_id(0)
    ng = pl.num_programs(0)
    n_tc = T // TC_CHUNK
    b = i // n_tc
    tci = i % n_tc
    slot = jax.lax.rem(i, NBUF)

    def copy_for(step):
        bb = step // n_tc
        tt = jax.lax.rem(step, n_tc)
        sl = jax.lax.rem(step, NBUF)
        return pltpu.make_async_copy(
            bufs.at[sl],
            out_ref.at[bb, pl.ds(tt * TC_CHUNK, TC_CHUNK)],
            sems.at[sl],
        )

    @pl.when(i >= NBUF)
    def _():
        copy_for(i - NBUF).wait()

    wt = w_ref[...]
    _fill(bufs.at[slot], wt, tci)
    copy_for(i).start()

    @pl.when(i == ng - 1)
    def _():
        for k in range(NBUF):
            step = ng - NBUF + k
            copy_for(step).wait()


def kernel(u, W):
    batch = u.shape[0]
    grid = (batch * (T // TC_CHUNK),)
    out = pl.pallas_call(
        _side_info_manual,
        grid=grid,
        in_specs=[pl.BlockSpec((EMB, N), lambda i: (0, 0))],
        out_specs=pl.BlockSpec(memory_space=pl.ANY),
        out_shape=jax.ShapeDtypeStruct((batch, T, OUT_DIM, N), jnp.float32),
        scratch_shapes=[
            pltpu.VMEM((NBUF, TC_CHUNK, OUT_DIM, N), jnp.float32),
            pltpu.SemaphoreType.DMA((NBUF,)),
        ],
        compiler_params=pltpu.CompilerParams(
            dimension_semantics=("arbitrary",),
        ),
    )(W.T)
    return jnp.transpose(out, (0, 1, 3, 2))


# speedup vs baseline: 1.3762x; 1.0184x over previous
import jax
import jax.numpy as jnp
from jax.experimental import pallas as pl
from jax.experimental.pallas import tpu as pltpu

T = 128
N = 512
EMB = 16
PE_DIM = 128
OUT_DIM = PE_DIM + EMB
TC_CHUNK = 16


def _side_info_block(wt_ref, out_ref):
    tci = pl.program_id(1)
    t0 = (tci * TC_CHUNK).astype(jnp.float32)
    c = jax.lax.broadcasted_iota(jnp.int32, (PE_DIM, TC_CHUNK), 0)
    t = t0 + jax.lax.broadcasted_iota(
        jnp.int32, (PE_DIM, TC_CHUNK), 1
    ).astype(jnp.float32)
    pair = (c // 2).astype(jnp.float32)
    div = jnp.exp(pair * (-2.0 * jnp.log(10000.0) / PE_DIM))
    ang = t * div
    val = jnp.where(c % 2 == 0, jnp.sin(ang), jnp.cos(ang))
    wt = wt_ref[...]
    for ti in range(TC_CHUNK):
        out_ref[0, ti, 0:PE_DIM, :] = jnp.broadcast_to(
            val[:, ti : ti + 1], (PE_DIM, N)
        )
        out_ref[0, ti, PE_DIM:OUT_DIM, :] = wt


def kernel(u, W):
    batch = u.shape[0]
    grid = (batch, T // TC_CHUNK)
    out = pl.pallas_call(
        _side_info_block,
        grid=grid,
        in_specs=[pl.BlockSpec((EMB, N), lambda b, tc: (0, 0))],
        out_specs=pl.BlockSpec(
            (1, TC_CHUNK, OUT_DIM, N), lambda b, tc: (b, tc, 0, 0)
        ),
        out_shape=jax.ShapeDtypeStruct((batch, T, OUT_DIM, N), jnp.float32),
        compiler_params=pltpu.CompilerParams(
            dimension_semantics=("parallel", "parallel"),
        ),
    )(W.T)
    return jnp.transpose(out, (0, 1, 3, 2))
